# Initial kernel scaffold; baseline (speedup 1.0000x reference)
import jax
import jax.numpy as jnp
from jax.experimental import pallas as pl


def kernel(x, edge_index, w_node, W1, b1, beta1, W2, b2, beta2):
    raise NotImplementedError("write your pallas kernel here")


# speedup vs baseline: 8.5225x; 8.5225x over previous
import functools

import jax
import jax.numpy as jnp
from jax import lax
from jax.experimental import pallas as pl
from jax.experimental.pallas import tpu as pltpu
from jax.experimental.pallas import tpu_sc as plsc

N = 50000
E = 800000
D_FEAT = 32
D = 64
EPS = 1e-3

NC = 2
NS = 16
HALF = D // 2

EPT = E // NS
CH = 400
NCHUNK = EPT // CH
RPT = N // NS

DCH = 10000
DNCH = EPT // DCH
ZCH = 3200

_mesh = plsc.VectorSubcoreMesh(core_axis_name="c", subcore_axis_name="s")


def _zero_rows(ref, nrows, ncols):
    zv = jnp.zeros((16,), jnp.float32)
    per_row = ncols // 16

    def z(i, _):
        r = i // per_row
        col = (i % per_row) * 16
        ref[r, pl.ds(col, 16)] = zv
        return 0

    lax.fori_loop(0, nrows * per_row, z, 0, unroll=4)


def _fill_1d(ref, n, val):
    v = jnp.full((16,), val, jnp.float32)

    def z(i, _):
        ref[pl.ds(i * 16, 16)] = v
        return 0

    lax.fori_loop(0, n // 16, z, 0, unroll=4)


NP = NS * ZCH


@functools.partial(
    pl.kernel,
    out_type=jax.ShapeDtypeStruct((2 * NP,), jnp.float32),
    mesh=_mesh,
    scratch_types=[
        pltpu.VMEM_SHARED((NS * ZCH,), jnp.float32),
        pltpu.VMEM((DCH,), jnp.int32),
        pltpu.VMEM((DCH,), jnp.float32),
        pltpu.VMEM((ZCH,), jnp.float32),
        pltpu.SemaphoreType.DMA,
    ],
)
def _deg_kernel(ei, deg_out, acc, idx, ones, zeros, sem):
    c = lax.axis_index("c")
    s = lax.axis_index("s")
    _fill_1d(ones, DCH, 1.0)
    _fill_1d(zeros, ZCH, 0.0)
    pltpu.sync_copy(zeros, acc.at[pl.ds(s * ZCH, ZCH)])
    plsc.subcore_barrier()

    ebase = s * EPT

    def step(i, _):
        off = ebase + i * DCH

        @pl.when(c == 0)
        def _():
            pltpu.sync_copy(ei.at[pl.ds(off, DCH)], idx)

        @pl.when(c == 1)
        def _():
            pltpu.sync_copy(ei.at[pl.ds(E + off, DCH)], idx)

        pltpu.sync_copy(ones, acc.at[idx], add=True)
        return 0

    lax.fori_loop(0, DNCH, step, 0)
    plsc.subcore_barrier()

    wbase = s * ZCH
    pltpu.sync_copy(acc.at[pl.ds(wbase, ZCH)], deg_out.at[pl.ds(c * NP + wbase, ZCH)])


@functools.partial(
    pl.kernel,
    out_type=(
        jax.ShapeDtypeStruct((N, HALF), jnp.float32),
        jax.ShapeDtypeStruct((N, HALF), jnp.float32),
    ),
    mesh=_mesh,
    compiler_params=pltpu.CompilerParams(use_tc_tiling_on_sc=False),
    scratch_types=[
        pltpu.VMEM_SHARED((N, HALF), jnp.float32),
        pltpu.VMEM((CH,), jnp.int32),
        pltpu.VMEM((CH,), jnp.int32),
        pltpu.VMEM((CH, HALF), jnp.float32),
        pltpu.SemaphoreType.DMA,
    ],
)
def _agg_kernel(fa, fb, ei, out_a, out_b, acc, sidx, didx, rows, sem):
    c = lax.axis_index("c")
    s = lax.axis_index("s")

    _zero_rows(rows, CH, HALF)
    rbase = s * ZCH
    zsrc = rows

    @pl.when(s < NS - 1)
    def _():
        for k in range(ZCH // 400):
            pltpu.sync_copy(zsrc, acc.at[pl.ds(rbase + k * 400, 400)])

    @pl.when(s == NS - 1)
    def _():
        for k in range((N - (NS - 1) * ZCH) // 400):
            pltpu.sync_copy(zsrc, acc.at[pl.ds(rbase + k * 400, 400)])

    plsc.subcore_barrier()

    ebase = s * EPT

    def step(i, _):
        off = ebase + i * CH
        pltpu.sync_copy(ei.at[pl.ds(off, CH)], sidx)
        pltpu.sync_copy(ei.at[pl.ds(E + off, CH)], didx)

        @pl.when(c == 0)
        def _():
            pltpu.async_copy(fa.at[sidx], rows, sem).wait()

        @pl.when(c == 1)
        def _():
            pltpu.async_copy(fb.at[sidx], rows, sem).wait()

        pltpu.sync_copy(rows, acc.at[didx], add=True)
        return 0

    lax.fori_loop(0, NCHUNK, step, 0)
    plsc.subcore_barrier()

    @pl.when(s < NS - 1)
    def _():
        @pl.when(c == 0)
        def _():
            pltpu.sync_copy(acc.at[pl.ds(rbase, ZCH)], out_a.at[pl.ds(rbase, ZCH)])

        @pl.when(c == 1)
        def _():
            pltpu.sync_copy(acc.at[pl.ds(rbase, ZCH)], out_b.at[pl.ds(rbase, ZCH)])

    @pl.when(s == NS - 1)
    def _():
        last = N - (NS - 1) * ZCH

        @pl.when(c == 0)
        def _():
            pltpu.sync_copy(acc.at[pl.ds(rbase, last)], out_a.at[pl.ds(rbase, last)])

        @pl.when(c == 1)
        def _():
            pltpu.sync_copy(acc.at[pl.ds(rbase, last)], out_b.at[pl.ds(rbase, last)])


BN = 1000
GRID = N // BN


def _norm(deg_row):
    return lax.rsqrt(jnp.maximum(deg_row, 1.0))


def _prep_body(x_ref, wn_ref, ds_ref, h_ref, fa_ref, fb_ref):
    xb = x_ref[...]
    wnb = jnp.broadcast_to(wn_ref[...], (BN, D_FEAT))
    ns = _norm(ds_ref[0, 0, :])[:, None]
    h_ref[:, :D_FEAT] = xb
    h_ref[:, D_FEAT:] = wnb
    fa_ref[...] = xb * ns
    fb_ref[...] = wnb * ns


def _dense_body(with_feat, aa_ref, ab_ref, ds_ref, dd_ref, h_ref, w_ref, b_ref,
                beta_ref, ho_ref, *feat_refs):
    agg = jnp.concatenate([aa_ref[...], ab_ref[...]], axis=1)
    nd = _norm(dd_ref[0, 0, :])[:, None]
    rst = jnp.dot(agg * nd, w_ref[...], preferred_element_type=jnp.float32)
    rst = jnp.maximum(rst + b_ref[...], 0.0)
    out = h_ref[...] + rst
    mean = jnp.mean(out, axis=1, keepdims=True)
    cent = out - mean
    var = jnp.mean(cent * cent, axis=1, keepdims=True)
    y = cent * lax.rsqrt(var + EPS) + beta_ref[...]
    ho_ref[...] = y
    if with_feat:
        fa_ref, fb_ref = feat_refs
        ns = _norm(ds_ref[0, 0, :])[:, None]
        fa_ref[...] = y[:, :HALF] * ns
        fb_ref[...] = y[:, HALF:] * ns


def _row_spec(cols):
    return pl.BlockSpec((BN, cols), lambda i: (i, 0))


def _full_spec(shape):
    ndims = len(shape)
    return pl.BlockSpec(shape, lambda i: (0,) * ndims)


_DEG_SRC_SPEC = pl.BlockSpec((1, 1, BN), lambda i: (i, 0, 0))
_DEG_DST_SPEC = pl.BlockSpec((1, 1, BN), lambda i: (GRID + i, 0, 0))


def _prep_call(x, w_node, deg3):
    return pl.pallas_call(
        _prep_body,
        grid=(GRID,),
        in_specs=[_row_spec(D_FEAT), _full_spec((1, D_FEAT)), _DEG_SRC_SPEC],
        out_specs=(_row_spec(D), _row_spec(HALF), _row_spec(HALF)),
        out_shape=(
            jax.ShapeDtypeStruct((N, D), jnp.float32),
            jax.ShapeDtypeStruct((N, HALF), jnp.float32),
            jax.ShapeDtypeStruct((N, HALF), jnp.float32),
        ),
    )(x, w_node, deg3)


def _dense_call(with_feat, aa, ab, deg3, h, w, b, beta):
    out_shape = [jax.ShapeDtypeStruct((N, D), jnp.float32)]
    out_specs = [_row_spec(D)]
    if with_feat:
        out_shape += [jax.ShapeDtypeStruct((N, HALF), jnp.float32)] * 2
        out_specs += [_row_spec(HALF)] * 2
    return pl.pallas_call(
        functools.partial(_dense_body, with_feat),
        grid=(GRID,),
        in_specs=[
            _row_spec(HALF), _row_spec(HALF), _DEG_SRC_SPEC, _DEG_DST_SPEC,
            _row_spec(D), _full_spec((D, D)), _full_spec((1, D)),
            _full_spec((1, D)),
        ],
        out_specs=tuple(out_specs),
        out_shape=tuple(out_shape),
    )(aa, ab, deg3, deg3, h, w, b, beta)


def kernel(x, edge_index, w_node, W1, b1, beta1, W2, b2, beta2):
    b1r = b1.reshape(1, D)
    beta1r = beta1.reshape(1, D)
    b2r = b2.reshape(1, D)
    beta2r = beta2.reshape(1, D)

    ei_flat = edge_index.reshape(2 * E)
    deg_pad = _deg_kernel(ei_flat)
    deg3 = deg_pad.reshape(2, NP)[:, :N].reshape(2 * GRID, 1, BN)
    h0, fa1, fb1 = _prep_call(x, w_node, deg3)
    agg1a, agg1b = _agg_kernel(fa1, fb1, ei_flat)
    h1, fa2, fb2 = _dense_call(True, agg1a, agg1b, deg3, h0, W1, b1r, beta1r)
    agg2a, agg2b = _agg_kernel(fa2, fb2, ei_flat)
    (h2,) = _dense_call(False, agg2a, agg2b, deg3, h1, W2, b2r, beta2r)
    return h2


# speedup vs baseline: 11.2314x; 1.3178x over previous
import functools

import jax
import jax.numpy as jnp
from jax import lax
from jax.experimental import pallas as pl
from jax.experimental.pallas import tpu as pltpu
from jax.experimental.pallas import tpu_sc as plsc

N = 50000
E = 800000
D_FEAT = 32
D = 64
EPS = 1e-3

NC = 2
NS = 16
HALF = D // 2

EPT = E // NS
CH = 400
NCHUNK = EPT // CH
RPT = N // NS

DCH = 10000
DNCH = EPT // DCH
ZCH = 3200

_mesh = plsc.VectorSubcoreMesh(core_axis_name="c", subcore_axis_name="s")


def _zero_rows(ref, nrows, ncols):
    zv = jnp.zeros((16,), jnp.float32)
    per_row = ncols // 16

    def z(i, _):
        r = i // per_row
        col = (i % per_row) * 16
        ref[r, pl.ds(col, 16)] = zv
        return 0

    lax.fori_loop(0, nrows * per_row, z, 0, unroll=4)


def _fill_1d(ref, n, val):
    v = jnp.full((16,), val, jnp.float32)

    def z(i, _):
        ref[pl.ds(i * 16, 16)] = v
        return 0

    lax.fori_loop(0, n // 16, z, 0, unroll=4)


NP = NS * ZCH


@functools.partial(
    pl.kernel,
    out_type=jax.ShapeDtypeStruct((2 * NP,), jnp.float32),
    mesh=_mesh,
    scratch_types=[
        pltpu.VMEM_SHARED((NS * ZCH,), jnp.float32),
        pltpu.VMEM((DCH,), jnp.int32),
        pltpu.VMEM((DCH,), jnp.float32),
        pltpu.VMEM((ZCH,), jnp.float32),
        pltpu.SemaphoreType.DMA,
    ],
)
def _deg_kernel(ei, deg_out, acc, idx, ones, zeros, sem):
    c = lax.axis_index("c")
    s = lax.axis_index("s")
    _fill_1d(ones, DCH, 1.0)
    _fill_1d(zeros, ZCH, 0.0)
    pltpu.sync_copy(zeros, acc.at[pl.ds(s * ZCH, ZCH)])
    plsc.subcore_barrier()

    ebase = s * EPT

    def step(i, _):
        off = ebase + i * DCH

        @pl.when(c == 0)
        def _():
            pltpu.sync_copy(ei.at[pl.ds(off, DCH)], idx)

        @pl.when(c == 1)
        def _():
            pltpu.sync_copy(ei.at[pl.ds(E + off, DCH)], idx)

        pltpu.sync_copy(ones, acc.at[idx], add=True)
        return 0

    lax.fori_loop(0, DNCH, step, 0)
    plsc.subcore_barrier()

    wbase = s * ZCH
    pltpu.sync_copy(acc.at[pl.ds(wbase, ZCH)], deg_out.at[pl.ds(c * NP + wbase, ZCH)])


@functools.partial(
    pl.kernel,
    out_type=(
        jax.ShapeDtypeStruct((N, HALF), jnp.float32),
        jax.ShapeDtypeStruct((N, HALF), jnp.float32),
    ),
    mesh=_mesh,
    compiler_params=pltpu.CompilerParams(use_tc_tiling_on_sc=False),
    scratch_types=[
        pltpu.VMEM_SHARED((N, HALF), jnp.float32),
        pltpu.VMEM((2, CH), jnp.int32),
        pltpu.VMEM((2, CH), jnp.int32),
        pltpu.VMEM((2, CH, HALF), jnp.float32),
        pltpu.SemaphoreType.DMA,
        pltpu.SemaphoreType.DMA,
        pltpu.SemaphoreType.DMA,
        pltpu.SemaphoreType.DMA,
    ],
)
def _agg_kernel(fa, fb, ei, out_a, out_b, acc, sidx, didx, rows,
                sg0, sg1, ss0, ss1):
    c = lax.axis_index("c")
    s = lax.axis_index("s")
    sem_g = (sg0, sg1)
    sem_s = (ss0, ss1)

    _zero_rows(rows.at[0], CH, HALF)
    rbase = s * ZCH
    zsrc = rows.at[0]

    @pl.when(s < NS - 1)
    def _():
        for k in range(ZCH // CH):
            pltpu.sync_copy(zsrc, acc.at[pl.ds(rbase + k * CH, CH)])

    @pl.when(s == NS - 1)
    def _():
        for k in range((N - (NS - 1) * ZCH) // CH):
            pltpu.sync_copy(zsrc, acc.at[pl.ds(rbase + k * CH, CH)])

    plsc.subcore_barrier()

    ebase = s * EPT

    def load_idx(b, k):
        off = ebase + k * CH
        pltpu.sync_copy(ei.at[pl.ds(off, CH)], sidx.at[b])
        pltpu.sync_copy(ei.at[pl.ds(E + off, CH)], didx.at[b])

    def start_gather(b):
        @pl.when(c == 0)
        def _():
            pltpu.async_copy(fa.at[sidx.at[b]], rows.at[b], sem_g[b])

        @pl.when(c == 1)
        def _():
            pltpu.async_copy(fb.at[sidx.at[b]], rows.at[b], sem_g[b])

    def wait_gather(b):
        @pl.when(c == 0)
        def _():
            pltpu.make_async_copy(fa.at[sidx.at[b]], rows.at[b], sem_g[b]).wait()

        @pl.when(c == 1)
        def _():
            pltpu.make_async_copy(fb.at[sidx.at[b]], rows.at[b], sem_g[b]).wait()

    def start_scatter(b):
        pltpu.async_copy(rows.at[b], acc.at[didx.at[b]], sem_s[b], add=True)

    def wait_scatter(b):
        pltpu.make_async_copy(rows.at[b], acc.at[didx.at[b]], sem_s[b]).wait()

    load_idx(0, 0)
    start_gather(0)

    def pair(i, _):
        for b in (0, 1):
            k = 2 * i + b
            nb = 1 - b

            @pl.when(k < NCHUNK - 1)
            def _():
                @pl.when(k >= 1)
                def _():
                    wait_scatter(nb)

                load_idx(nb, k + 1)
                start_gather(nb)

            wait_gather(b)
            start_scatter(b)
        return 0

    lax.fori_loop(0, NCHUNK // 2, pair, 0)
    wait_gather(0)
    start_scatter(0)
    wait_scatter(1)
    wait_scatter(0)
    plsc.subcore_barrier()

    @pl.when(s < NS - 1)
    def _():
        @pl.when(c == 0)
        def _():
            pltpu.sync_copy(acc.at[pl.ds(rbase, ZCH)], out_a.at[pl.ds(rbase, ZCH)])

        @pl.when(c == 1)
        def _():
            pltpu.sync_copy(acc.at[pl.ds(rbase, ZCH)], out_b.at[pl.ds(rbase, ZCH)])

    @pl.when(s == NS - 1)
    def _():
        last = N - (NS - 1) * ZCH

        @pl.when(c == 0)
        def _():
            pltpu.sync_copy(acc.at[pl.ds(rbase, last)], out_a.at[pl.ds(rbase, last)])

        @pl.when(c == 1)
        def _():
            pltpu.sync_copy(acc.at[pl.ds(rbase, last)], out_b.at[pl.ds(rbase, last)])


BN = 1000
GRID = N // BN


def _norm(deg_row):
    return lax.rsqrt(jnp.maximum(deg_row, 1.0))


def _prep_body(x_ref, wn_ref, ds_ref, h_ref, fa_ref, fb_ref):
    xb = x_ref[...]
    wnb = jnp.broadcast_to(wn_ref[...], (BN, D_FEAT))
    ns = _norm(ds_ref[0, 0, :])[:, None]
    h_ref[:, :D_FEAT] = xb
    h_ref[:, D_FEAT:] = wnb
    fa_ref[...] = xb * ns
    fb_ref[...] = wnb * ns


def _dense_body(with_feat, aa_ref, ab_ref, ds_ref, dd_ref, h_ref, w_ref, b_ref,
                beta_ref, ho_ref, *feat_refs):
    agg = jnp.concatenate([aa_ref[...], ab_ref[...]], axis=1)
    nd = _norm(dd_ref[0, 0, :])[:, None]
    rst = jnp.dot(agg * nd, w_ref[...], preferred_element_type=jnp.float32)
    rst = jnp.maximum(rst + b_ref[...], 0.0)
    out = h_ref[...] + rst
    mean = jnp.mean(out, axis=1, keepdims=True)
    cent = out - mean
    var = jnp.mean(cent * cent, axis=1, keepdims=True)
    y = cent * lax.rsqrt(var + EPS) + beta_ref[...]
    ho_ref[...] = y
    if with_feat:
        fa_ref, fb_ref = feat_refs
        ns = _norm(ds_ref[0, 0, :])[:, None]
        fa_ref[...] = y[:, :HALF] * ns
        fb_ref[...] = y[:, HALF:] * ns


def _row_spec(cols):
    return pl.BlockSpec((BN, cols), lambda i: (i, 0))


def _full_spec(shape):
    ndims = len(shape)
    return pl.BlockSpec(shape, lambda i: (0,) * ndims)


_DEG_SRC_SPEC = pl.BlockSpec((1, 1, BN), lambda i: (i, 0, 0))
_DEG_DST_SPEC = pl.BlockSpec((1, 1, BN), lambda i: (GRID + i, 0, 0))


def _prep_call(x, w_node, deg3):
    return pl.pallas_call(
        _prep_body,
        grid=(GRID,),
        in_specs=[_row_spec(D_FEAT), _full_spec((1, D_FEAT)), _DEG_SRC_SPEC],
        out_specs=(_row_spec(D), _row_spec(HALF), _row_spec(HALF)),
        out_shape=(
            jax.ShapeDtypeStruct((N, D), jnp.float32),
            jax.ShapeDtypeStruct((N, HALF), jnp.float32),
            jax.ShapeDtypeStruct((N, HALF), jnp.float32),
        ),
    )(x, w_node, deg3)


def _dense_call(with_feat, aa, ab, deg3, h, w, b, beta):
    out_shape = [jax.ShapeDtypeStruct((N, D), jnp.float32)]
    out_specs = [_row_spec(D)]
    if with_feat:
        out_shape += [jax.ShapeDtypeStruct((N, HALF), jnp.float32)] * 2
        out_specs += [_row_spec(HALF)] * 2
    return pl.pallas_call(
        functools.partial(_dense_body, with_feat),
        grid=(GRID,),
        in_specs=[
            _row_spec(HALF), _row_spec(HALF), _DEG_SRC_SPEC, _DEG_DST_SPEC,
            _row_spec(D), _full_spec((D, D)), _full_spec((1, D)),
            _full_spec((1, D)),
        ],
        out_specs=tuple(out_specs),
        out_shape=tuple(out_shape),
    )(aa, ab, deg3, deg3, h, w, b, beta)


def kernel(x, edge_index, w_node, W1, b1, beta1, W2, b2, beta2):
    b1r = b1.reshape(1, D)
    beta1r = beta1.reshape(1, D)
    b2r = b2.reshape(1, D)
    beta2r = beta2.reshape(1, D)

    ei_flat = edge_index.reshape(2 * E)
    deg_pad = _deg_kernel(ei_flat)
    deg3 = deg_pad.reshape(2, NP)[:, :N].reshape(2 * GRID, 1, BN)
    h0, fa1, fb1 = _prep_call(x, w_node, deg3)
    agg1a, agg1b = _agg_kernel(fa1, fb1, ei_flat)
    h1, fa2, fb2 = _dense_call(True, agg1a, agg1b, deg3, h0, W1, b1r, beta1r)
    agg2a, agg2b = _agg_kernel(fa2, fb2, ei_flat)
    (h2,) = _dense_call(False, agg2a, agg2b, deg3, h1, W2, b2r, beta2r)
    return h2
